# baseline (speedup 1.0000x reference)
import functools

import jax
import jax.numpy as jnp
from jax import lax
from jax.experimental import pallas as pl
from jax.experimental.pallas import tpu as pltpu
from jax.experimental.pallas import tpu_sc as plsc

_T = 100000
_J = 29
_NB = 30
_B = 4096
_OBS = 480
_H1, _H2, _H3 = 512, 256, 128
_ACT = 29

_NC, _NS, _L = 2, 16, 16
_NW = _NC * _NS
_BPW = _B // _NW

_WIDTHS = (_J, _J, _NB * 3, _NB * 4, _NB * 3, _NB * 3)
_NCH = tuple((w + 30) // 16 for w in _WIDTHS)
_CHUNKS = tuple(_T * w // _L for w in _WIDTHS)
_HB = 2
_HR = _BPW // _HB
_NG = _HR // _L

def _gather_body(ts_hbm, t0, t1, t2, t3, t4, t5,
                 o0, o1, o2, o3, o4, o5,
                 idx_v, ib0, ib1, ib2, ib3, ib4, ib5,
                 cb0, cb1, cb2, cb3, cb4, cb5,
                 ob0, ob1, ob2, ob3, ob4, ob5, sem):
    wid = lax.axis_index("s") * _NC + lax.axis_index("c")
    base = wid * _BPW
    tabs = (t0, t1, t2, t3, t4, t5)
    outs = (o0, o1, o2, o3, o4, o5)
    idxb = (ib0, ib1, ib2, ib3, ib4, ib5)
    chkb = (cb0, cb1, cb2, cb3, cb4, cb5)
    outb = (ob0, ob1, ob2, ob3, ob4, ob5)
    pltpu.sync_copy(ts_hbm.at[pl.ds(base, _BPW)], idx_v)
    _IOTA = lax.iota(jnp.int32, _L)

    def half(h, _):
        def build(g, _):
            t = jnp.minimum(idx_v[pl.ds(h * _HR + g * _L, _L)], _T - 1)
            pos0 = g * (_L * 1)
            for k in range(6):
                nch, w = _NCH[k], _WIDTHS[k]
                c0 = (t * w) >> 4
                p_base = (g * _L * nch) + _IOTA * nch
                for m in range(nch):
                    c = jnp.minimum(c0 + m, _CHUNKS[k] - 1)
                    plsc.store_scatter(idxb[k], [p_base + m], c)
            return ()

        lax.fori_loop(0, _NG, build, (), unroll=False)
        for k in range(6):
            pltpu.async_copy(tabs[k].at[idxb[k]], chkb[k], sem)
        for k in range(6):
            pltpu.make_async_copy(tabs[k].at[pl.ds(0, _HR * _NCH[k])],
                                  chkb[k], sem).wait()

        def extract(g, _):
            t = jnp.minimum(idx_v[pl.ds(h * _HR + g * _L, _L)], _T - 1)
            rloc = g * _L + _IOTA
            for k in range(6):
                nch, w = _NCH[k], _WIDTHS[k]
                s = (t * w) & 15
                rowbase = rloc * nch
                for j in range(w):
                    wv = s + j
                    vals = plsc.load_gather(
                        chkb[k], [rowbase + (wv >> 4), wv & 15])
                    plsc.store_scatter(
                        outb[k], [rloc, jnp.full((_L,), j, jnp.int32)], vals)
            return ()

        lax.fori_loop(0, _NG, extract, (), unroll=False)
        for k in range(6):
            pltpu.sync_copy(outb[k],
                            outs[k].at[pl.ds(base + h * _HR, _HR)])
        return ()

    lax.fori_loop(0, _HB, half, (), unroll=False)


_gather = pl.kernel(
    _gather_body,
    out_type=tuple(jax.ShapeDtypeStruct((_B, w), jnp.float32) for w in _WIDTHS),
    mesh=plsc.VectorSubcoreMesh(core_axis_name="c", subcore_axis_name="s"),
    scratch_types=[pltpu.VMEM((_BPW,), jnp.int32)]
    + [pltpu.VMEM((_HR * n,), jnp.int32) for n in _NCH]
    + [pltpu.VMEM((_HR * n, _L), jnp.float32) for n in _NCH]
    + [pltpu.VMEM((_HR, w), jnp.float32) for w in _WIDTHS]
    + [pltpu.SemaphoreType.DMA],
    compiler_params=pltpu.CompilerParams(use_tc_tiling_on_sc=False,
                                         needs_layout_passes=False),
)


def _elu(x):
    return jnp.where(x > 0, x, jnp.exp(jnp.minimum(x, 0.0)) - 1.0)


def _mlp_body(obs_ref, w1, b1, w2, b2, w3, b3, w4, b4, out_ref):
    h = _elu(jnp.dot(obs_ref[...], w1[...], preferred_element_type=jnp.float32)
             + b1[...])
    h = _elu(jnp.dot(h, w2[...], preferred_element_type=jnp.float32) + b2[...])
    h = _elu(jnp.dot(h, w3[...], preferred_element_type=jnp.float32) + b3[...])
    out_ref[...] = (jnp.dot(h, w4[...], preferred_element_type=jnp.float32)
                    + b4[...])


_BM = 512


def _mlp(obs, W1, b1, W2, b2, W3, b3, W4, b4):
    full = lambda r, c: pl.BlockSpec((r, c), lambda i: (0, 0))
    return pl.pallas_call(
        _mlp_body,
        grid=(_B // _BM,),
        in_specs=[
            pl.BlockSpec((_BM, _OBS), lambda i: (i, 0)),
            full(_OBS, _H1), full(1, _H1),
            full(_H1, _H2), full(1, _H2),
            full(_H2, _H3), full(1, _H3),
            full(_H3, _ACT), full(1, _ACT),
        ],
        out_specs=pl.BlockSpec((_BM, _ACT), lambda i: (i, 0)),
        out_shape=jax.ShapeDtypeStruct((_B, _ACT), jnp.float32),
    )(obs, W1, b1.reshape(1, _H1), W2, b2.reshape(1, _H2),
      W3, b3.reshape(1, _H3), W4, b4.reshape(1, _ACT))


def kernel(obs, W1, b1, W2, b2, W3, b3, W4, b4, joint_pos, joint_vel,
           body_pos_w, body_quat_w, body_lin_vel_w, body_ang_vel_w, time_step):
    ts = time_step.reshape(_B).astype(jnp.int32)
    g = _gather(ts,
                joint_pos.reshape(_CHUNKS[0], _L),
                joint_vel.reshape(_CHUNKS[1], _L),
                body_pos_w.reshape(_CHUNKS[2], _L),
                body_quat_w.reshape(_CHUNKS[3], _L),
                body_lin_vel_w.reshape(_CHUNKS[4], _L),
                body_ang_vel_w.reshape(_CHUNKS[5], _L))
    policy_out = _mlp(obs, W1, b1, W2, b2, W3, b3, W4, b4)
    return (policy_out,
            g[0], g[1],
            g[2].reshape(_B, _NB, 3),
            g[3].reshape(_B, _NB, 4),
            g[4].reshape(_B, _NB, 3),
            g[5].reshape(_B, _NB, 3))


# speedup vs baseline: 13.5243x; 13.5243x over previous
import functools

import jax
import jax.numpy as jnp
from jax import lax
from jax.experimental import pallas as pl
from jax.experimental.pallas import tpu as pltpu
from jax.experimental.pallas import tpu_sc as plsc

_T = 100000
_J = 29
_NB = 30
_B = 4096
_OBS = 480
_H1, _H2, _H3 = 512, 256, 128
_ACT = 29

_NC, _NS, _L = 2, 16, 16
_NW = _NC * _NS
_BPW = _B // _NW

_WIDTHS = (_J, _J, _NB * 3, _NB * 4, _NB * 3, _NB * 3)

def _gather_body(ts_hbm, t0, t1, t2, t3, t4, t5,
                 o0, o1, o2, o3, o4, o5,
                 idx_v, r0, r1, r2, r3, r4, r5, sem):
    wid = lax.axis_index("s") * _NC + lax.axis_index("c")
    base = wid * _BPW
    tabs = (t0, t1, t2, t3, t4, t5)
    outs = (o0, o1, o2, o3, o4, o5)
    rows = (r0, r1, r2, r3, r4, r5)
    pltpu.sync_copy(ts_hbm.at[pl.ds(base, _BPW)], idx_v)
    iota = lax.iota(jnp.int32, _L)

    def group(g, _):
        vec = jnp.minimum(idx_v[pl.ds(g * _L, _L)], _T - 1)
        for i in range(_L):
            t = lax.reduce_max(jnp.where(iota == i, vec, 0), (0,))
            r = g * _L + i
            for tab, rv in zip(tabs, rows):
                pltpu.async_copy(tab.at[pl.ds(t, 1)], rv.at[pl.ds(r, 1)],
                                 sem)
        return ()

    lax.fori_loop(0, _BPW // _L, group, (), unroll=False)
    for tab, rv in zip(tabs, rows):
        pltpu.make_async_copy(tab.at[pl.ds(0, _BPW)], rv, sem).wait()
    for rv, o in zip(rows, outs):
        pltpu.sync_copy(rv, o.at[pl.ds(base, _BPW)])


_gather = pl.kernel(
    _gather_body,
    out_type=tuple(jax.ShapeDtypeStruct((_B, w), jnp.float32)
                   for w in _WIDTHS),
    mesh=plsc.VectorSubcoreMesh(core_axis_name="c", subcore_axis_name="s"),
    scratch_types=[pltpu.VMEM((_BPW,), jnp.int32)]
    + [pltpu.VMEM((_BPW, w), jnp.float32) for w in _WIDTHS]
    + [pltpu.SemaphoreType.DMA],
    compiler_params=pltpu.CompilerParams(use_tc_tiling_on_sc=False,
                                         needs_layout_passes=False),
)


def _elu(x):
    return jnp.where(x > 0, x, jnp.exp(jnp.minimum(x, 0.0)) - 1.0)


def _mlp_body(obs_ref, w1, b1, w2, b2, w3, b3, w4, b4, out_ref):
    h = _elu(jnp.dot(obs_ref[...], w1[...], preferred_element_type=jnp.float32)
             + b1[...])
    h = _elu(jnp.dot(h, w2[...], preferred_element_type=jnp.float32) + b2[...])
    h = _elu(jnp.dot(h, w3[...], preferred_element_type=jnp.float32) + b3[...])
    out_ref[...] = (jnp.dot(h, w4[...], preferred_element_type=jnp.float32)
                    + b4[...])


_BM = 512


def _mlp(obs, W1, b1, W2, b2, W3, b3, W4, b4):
    full = lambda r, c: pl.BlockSpec((r, c), lambda i: (0, 0))
    return pl.pallas_call(
        _mlp_body,
        grid=(_B // _BM,),
        in_specs=[
            pl.BlockSpec((_BM, _OBS), lambda i: (i, 0)),
            full(_OBS, _H1), full(1, _H1),
            full(_H1, _H2), full(1, _H2),
            full(_H2, _H3), full(1, _H3),
            full(_H3, _ACT), full(1, _ACT),
        ],
        out_specs=pl.BlockSpec((_BM, _ACT), lambda i: (i, 0)),
        out_shape=jax.ShapeDtypeStruct((_B, _ACT), jnp.float32),
    )(obs, W1, b1.reshape(1, _H1), W2, b2.reshape(1, _H2),
      W3, b3.reshape(1, _H3), W4, b4.reshape(1, _ACT))


def kernel(obs, W1, b1, W2, b2, W3, b3, W4, b4, joint_pos, joint_vel,
           body_pos_w, body_quat_w, body_lin_vel_w, body_ang_vel_w, time_step):
    ts = time_step.reshape(_B).astype(jnp.int32)
    g = _gather(ts, joint_pos, joint_vel,
                body_pos_w.reshape(_T, _NB * 3),
                body_quat_w.reshape(_T, _NB * 4),
                body_lin_vel_w.reshape(_T, _NB * 3),
                body_ang_vel_w.reshape(_T, _NB * 3))
    policy_out = _mlp(obs, W1, b1, W2, b2, W3, b3, W4, b4)
    return (policy_out,
            g[0], g[1],
            g[2].reshape(_B, _NB, 3),
            g[3].reshape(_B, _NB, 4),
            g[4].reshape(_B, _NB, 3),
            g[5].reshape(_B, _NB, 3))


# speedup vs baseline: 24.9572x; 1.8454x over previous
import functools

import jax
import jax.numpy as jnp
from jax import lax
from jax.experimental import pallas as pl
from jax.experimental.pallas import tpu as pltpu
from jax.experimental.pallas import tpu_sc as plsc

_T = 100000
_J = 29
_NB = 30
_B = 4096
_OBS = 480
_H1, _H2, _H3 = 512, 256, 128
_ACT = 29

_NC, _NS, _L = 2, 16, 16
_NW = _NC * _NS
_BPW = _B // _NW

_WIDTHS = (_J, _J, _NB * 3, _NB * 4, _NB * 3, _NB * 3)

def _gather_body(ts_hbm, t0, t1, t2, t3, t4, t5,
                 o0, o1, o2, o3, o4, o5,
                 idx_v, r0, r1, r2, r3, r4, r5, sem):
    wid = lax.axis_index("s") * _NC + lax.axis_index("c")
    base = wid * _BPW
    tabs = (t0, t1, t2, t3, t4, t5)
    outs = (o0, o1, o2, o3, o4, o5)
    rows = (r0, r1, r2, r3, r4, r5)
    pltpu.sync_copy(ts_hbm.at[pl.ds(base, _BPW)], idx_v)
    iota = lax.iota(jnp.int32, _L)

    def group(g, _):
        vec = jnp.minimum(idx_v[pl.ds(g * _L, _L)], _T - 1)
        for i in range(_L):
            t = lax.reduce_max(jnp.where(iota == i, vec, 0), (0,))
            r = g * _L + i
            for tab, rv in zip(tabs, rows):
                pltpu.async_copy(tab.at[pl.ds(t, 1)], rv.at[pl.ds(r, 1)],
                                 sem)
        return ()

    lax.fori_loop(0, _BPW // _L, group, (), unroll=False)
    for tab, rv in zip(tabs, rows):
        pltpu.make_async_copy(tab.at[pl.ds(0, _BPW)], rv, sem).wait()
    for rv, o in zip(rows, outs):
        pltpu.sync_copy(rv, o.at[pl.ds(base, _BPW)])


_gather = pl.kernel(
    _gather_body,
    out_type=tuple(jax.ShapeDtypeStruct((_B, w), jnp.float32)
                   for w in _WIDTHS),
    mesh=plsc.VectorSubcoreMesh(core_axis_name="c", subcore_axis_name="s"),
    scratch_types=[pltpu.VMEM((_BPW,), jnp.int32)]
    + [pltpu.VMEM((_BPW, w), jnp.float32) for w in _WIDTHS]
    + [pltpu.SemaphoreType.DMA],
    compiler_params=pltpu.CompilerParams(needs_layout_passes=False),
)


def _elu(x):
    return jnp.where(x > 0, x, jnp.exp(jnp.minimum(x, 0.0)) - 1.0)


def _mlp_body(obs_ref, w1, b1, w2, b2, w3, b3, w4, b4, out_ref):
    h = _elu(jnp.dot(obs_ref[...], w1[...], preferred_element_type=jnp.float32)
             + b1[...])
    h = _elu(jnp.dot(h, w2[...], preferred_element_type=jnp.float32) + b2[...])
    h = _elu(jnp.dot(h, w3[...], preferred_element_type=jnp.float32) + b3[...])
    out_ref[...] = (jnp.dot(h, w4[...], preferred_element_type=jnp.float32)
                    + b4[...])


_BM = 512


def _mlp(obs, W1, b1, W2, b2, W3, b3, W4, b4):
    full = lambda r, c: pl.BlockSpec((r, c), lambda i: (0, 0))
    return pl.pallas_call(
        _mlp_body,
        grid=(_B // _BM,),
        in_specs=[
            pl.BlockSpec((_BM, _OBS), lambda i: (i, 0)),
            full(_OBS, _H1), full(1, _H1),
            full(_H1, _H2), full(1, _H2),
            full(_H2, _H3), full(1, _H3),
            full(_H3, _ACT), full(1, _ACT),
        ],
        out_specs=pl.BlockSpec((_BM, _ACT), lambda i: (i, 0)),
        out_shape=jax.ShapeDtypeStruct((_B, _ACT), jnp.float32),
    )(obs, W1, b1.reshape(1, _H1), W2, b2.reshape(1, _H2),
      W3, b3.reshape(1, _H3), W4, b4.reshape(1, _ACT))


def kernel(obs, W1, b1, W2, b2, W3, b3, W4, b4, joint_pos, joint_vel,
           body_pos_w, body_quat_w, body_lin_vel_w, body_ang_vel_w, time_step):
    ts = time_step.reshape(_B).astype(jnp.int32)
    g = _gather(ts, joint_pos, joint_vel,
                body_pos_w.reshape(_T, _NB * 3),
                body_quat_w.reshape(_T, _NB * 4),
                body_lin_vel_w.reshape(_T, _NB * 3),
                body_ang_vel_w.reshape(_T, _NB * 3))
    policy_out = _mlp(obs, W1, b1, W2, b2, W3, b3, W4, b4)
    return (policy_out,
            g[0], g[1],
            g[2].reshape(_B, _NB, 3),
            g[3].reshape(_B, _NB, 4),
            g[4].reshape(_B, _NB, 3),
            g[5].reshape(_B, _NB, 3))
